# baseline (speedup 1.0000x reference)
import functools

import jax
import jax.numpy as jnp
from jax import lax
from jax.experimental import pallas as pl
from jax.experimental.pallas import tpu as pltpu
from jax.experimental.pallas import tpu_sc as plsc

_B = 4096
_V = 100000
_D = 64
_W = 5
_NEG = 5
_NW = 32
_BPW = _B // _NW
_NF = 2 * _W + 2 + 2 * _NEG
_DP = _D // 2
_NG = _BPW // 16
_GCOLS = list(range(2 * _W)) + [2 * _W + 1] + list(range(2 * _W + 2,
                                                         2 * _W + 2 + _NEG))


def _unpack2(pk):
    v = plsc.bitcast(pk, jnp.bfloat16)
    return plsc.unpack(v, format=plsc.PackFormat.INTERLEAVED)


def _sc_body(data_hbm, emb0_hbm, emb1_hbm, out_hbm,
             d22, idxs, acc, rb0, rb1, wb, nb0, nb1, nb2, nb3, nb4,
             linv_v, mask_v, res_v,
             s_acc, s_r0, s_r1, s_w, s_n0, s_n1, s_n2, s_n3, s_n4):
    wid = lax.axis_index("s") * 2 + lax.axis_index("c")
    base = wid * _BPW
    lane = lax.iota(jnp.int32, 16)

    pltpu.sync_copy(data_hbm.at[pl.ds(base, _BPW)], d22)

    for f, col in enumerate(_GCOLS):
        cvec = jnp.full((16,), col, jnp.int32)
        for g in range(_NG):
            v = plsc.load_gather(d22, [lane + g * 16, cvec])
            idxs[f, pl.ds(g * 16, 16)] = v

    ring = [rb0, rb1]
    ring_sems = [s_r0, s_r1]
    nbufs = [nb0, nb1, nb2, nb3, nb4]
    nsems = [s_n0, s_n1, s_n2, s_n3, s_n4]

    cps = {
        0: pltpu.async_copy(emb0_hbm.at[idxs.at[0]], rb0, s_r0),
        1: pltpu.async_copy(emb0_hbm.at[idxs.at[1]], rb1, s_r1),
    }
    cw = pltpu.async_copy(emb1_hbm.at[idxs.at[2 * _W]], wb, s_w)
    cns = [
        pltpu.async_copy(emb1_hbm.at[idxs.at[2 * _W + 1 + n]],
                         nbufs[n], nsems[n])
        for n in range(_NEG)
    ]

    lcol = jnp.full((16,), 2 * _W, jnp.int32)
    one = jnp.full((16,), 1.0, jnp.float32)
    for g in range(_NG):
        lv = plsc.load_gather(d22, [lane + g * 16, lcol])
        linv_v[pl.ds(g * 16, 16)] = one / lv.astype(jnp.float32)
    for n in range(_NEG):
        mcol = jnp.full((16,), 2 * _W + 2 + _NEG + n, jnp.int32)
        for g in range(_NG):
            mv = plsc.load_gather(d22, [lane + g * 16, mcol])
            mask_v[pl.ds(n * _BPW + g * 16, 16)] = mv.astype(jnp.float32)

    for j in range(2 * _W):
        slot = j % 2
        buf = ring[slot]
        cps[j].wait()

        def add_body(i, _, buf=buf, j=j):
            for c in range(_DP // 16):
                lo, hi = _unpack2(buf[i, pl.ds(c * 16, 16)])
                if j == 0:
                    acc[i, pl.ds(c * 32, 16)] = lo
                    acc[i, pl.ds(c * 32 + 16, 16)] = hi
                else:
                    plsc.addupdate(acc.at[i, pl.ds(c * 32, 16)], lo)
                    plsc.addupdate(acc.at[i, pl.ds(c * 32 + 16, 16)], hi)
            return 0

        lax.fori_loop(0, _BPW, add_body, 0)
        nxt = j + 2
        if nxt < 2 * _W:
            cps[nxt] = pltpu.async_copy(
                emb0_hbm.at[idxs.at[nxt]], buf, ring_sems[slot])

    cw.wait()
    for c in cns:
        c.wait()

    zeros = jnp.zeros((16,), jnp.float32)
    tbufs = [wb] + nbufs
    for g in range(_NG):
        bidx = lane + (g * 16)

        def make_step(coff):
            def dot_step(p, carry):
                pvec = jnp.full((16,), p, jnp.int32)
                a_lo = plsc.load_gather(acc, [bidx, pvec + coff])
                a_hi = plsc.load_gather(acc, [bidx, pvec + (coff + 16)])
                out = []
                for t in range(1 + _NEG):
                    lo, hi = _unpack2(
                        plsc.load_gather(tbufs[t], [bidx, pvec]))
                    out.append(carry[t] + a_lo * lo + a_hi * hi)
                return tuple(out)
            return dot_step

        res = lax.fori_loop(0, 16, make_step(0), (zeros,) * (1 + _NEG))
        res = lax.fori_loop(16, 32, make_step(16), res)

        linv = linv_v[pl.ds(g * 16, 16)]
        res_v[pl.ds(g * 16, 16)] = res[0] * linv
        for n in range(_NEG):
            m = mask_v[pl.ds(n * _BPW + g * 16, 16)]
            res_v[pl.ds((1 + n) * _BPW + g * 16, 16)] = res[1 + n] * linv * m

    pltpu.sync_copy(res_v.at[pl.ds(0, _BPW)], out_hbm.at[0, wid])
    for n in range(_NEG):
        pltpu.sync_copy(res_v.at[pl.ds((1 + n) * _BPW, _BPW)],
                        out_hbm.at[1 + n, wid])
        pltpu.sync_copy(mask_v.at[pl.ds(n * _BPW, _BPW)],
                        out_hbm.at[1 + _NEG + n, wid])


_sc_kernel = functools.partial(
    pl.kernel,
    out_type=jax.ShapeDtypeStruct((1 + 2 * _NEG, _NW, _BPW), jnp.float32),
    mesh=plsc.VectorSubcoreMesh(core_axis_name="c", subcore_axis_name="s"),
    compiler_params=pltpu.CompilerParams(
        needs_layout_passes=False, use_tc_tiling_on_sc=False),
    scratch_types=[
        pltpu.VMEM((_BPW, _NF), jnp.int32),
        pltpu.VMEM((2 * _W + 1 + _NEG, _BPW), jnp.int32),
        pltpu.VMEM((_BPW, _D), jnp.float32),
        *[pltpu.VMEM((_BPW, _DP), jnp.int32) for _ in range(8)],
        pltpu.VMEM((_BPW,), jnp.float32),
        pltpu.VMEM((_NEG * _BPW,), jnp.float32),
        pltpu.VMEM(((1 + _NEG) * _BPW,), jnp.float32),
        *[pltpu.SemaphoreType.DMA for _ in range(9)],
    ],
)(_sc_body)


def _loss_body(s_ref, o_ref):
    x = jnp.clip(s_ref[0], -10.0, 10.0)
    total = jnp.sum(jnp.log(1.0 + jnp.exp(-x)))
    for n in range(_NEG):
        z = jnp.clip(-s_ref[1 + n], -10.0, 10.0)
        total = total + jnp.sum(jnp.log(1.0 + jnp.exp(-z))
                                * s_ref[1 + _NEG + n])
    o_ref[...] = jnp.reshape(total, (1, 1))


def _packed(emb):
    h = emb.astype(jnp.bfloat16).reshape(emb.shape[0], _DP, 2)
    return lax.bitcast_convert_type(h, jnp.int32)


def kernel(data, emb0, emb1):
    raw = _sc_kernel(data, _packed(emb0), _packed(emb1))
    loss = pl.pallas_call(
        _loss_body,
        out_shape=jax.ShapeDtypeStruct((1, 1), jnp.float32),
    )(raw)
    return loss[0, 0]


# speedup vs baseline: 3.1539x; 3.1539x over previous
import functools

import jax
import jax.numpy as jnp
from jax import lax
from jax.experimental import pallas as pl
from jax.experimental.pallas import tpu as pltpu
from jax.experimental.pallas import tpu_sc as plsc

_B = 4096
_V = 100000
_D = 64
_W = 5
_NEG = 5
_NW = 32
_BPW = _B // _NW
_NF = 2 * _W + 2 + 2 * _NEG
_NCH = _D // 16
_NG = _BPW // 16
_GCOLS = list(range(2 * _W)) + [2 * _W + 1] + list(range(2 * _W + 2,
                                                         2 * _W + 2 + _NEG))


def _sc_body(data_hbm, emb0_hbm, emb1_hbm, out_hbm,
             d22, idxs, acc, rb0, rb1, rb2, rb3, wb, nb0, nb1, nb2, nb3, nb4,
             linv_v, mask_v, res_v,
             s_acc, s_r0, s_r1, s_r2, s_r3, s_w, s_n0, s_n1, s_n2, s_n3, s_n4):
    wid = lax.axis_index("s") * 2 + lax.axis_index("c")
    base = wid * _BPW
    lane = lax.iota(jnp.int32, 16)

    pltpu.sync_copy(data_hbm.at[pl.ds(base, _BPW)], d22)

    for f, col in enumerate(_GCOLS):
        cvec = jnp.full((16,), col, jnp.int32)
        for g in range(_NG):
            v = plsc.load_gather(d22, [lane + g * 16, cvec])
            idxs[f, pl.ds(g * 16, 16)] = v

    ring = [rb0, rb1, rb2, rb3]
    ring_sems = [s_r0, s_r1, s_r2, s_r3]
    nbufs = [nb0, nb1, nb2, nb3, nb4]
    nsems = [s_n0, s_n1, s_n2, s_n3, s_n4]

    cps = {0: pltpu.async_copy(emb0_hbm.at[idxs.at[0]], acc, s_acc)}
    for j in range(1, 5):
        cps[j] = pltpu.async_copy(
            emb0_hbm.at[idxs.at[j]], ring[j - 1], ring_sems[j - 1])
    cw = pltpu.async_copy(emb1_hbm.at[idxs.at[2 * _W]], wb, s_w)
    cns = [
        pltpu.async_copy(emb1_hbm.at[idxs.at[2 * _W + 1 + n]],
                         nbufs[n], nsems[n])
        for n in range(_NEG)
    ]

    lcol = jnp.full((16,), 2 * _W, jnp.int32)
    one = jnp.full((16,), 1.0, jnp.float32)
    for g in range(_NG):
        lv = plsc.load_gather(d22, [lane + g * 16, lcol])
        linv_v[pl.ds(g * 16, 16)] = one / lv.astype(jnp.float32)
    for n in range(_NEG):
        mcol = jnp.full((16,), 2 * _W + 2 + _NEG + n, jnp.int32)
        for g in range(_NG):
            mv = plsc.load_gather(d22, [lane + g * 16, mcol])
            mask_v[pl.ds(n * _BPW + g * 16, 16)] = mv.astype(jnp.float32)

    cps[0].wait()
    for j in range(1, 2 * _W):
        slot = (j - 1) % 4
        buf = ring[slot]
        cps[j].wait()

        def add_body(i, _, buf=buf):
            for c in range(_NCH):
                sl = pl.ds(c * 16, 16)
                plsc.addupdate(acc.at[i, sl], buf[i, sl])
            return 0

        lax.fori_loop(0, _BPW, add_body, 0)
        nxt = j + 4
        if nxt < 2 * _W:
            cps[nxt] = pltpu.async_copy(
                emb0_hbm.at[idxs.at[nxt]], buf, ring_sems[slot])

    cw.wait()
    for c in cns:
        c.wait()

    zeros = jnp.zeros((16,), jnp.float32)
    for g in range(_NG):
        bidx = lane + (g * 16)

        def dot_step(d, carry):
            dvec = jnp.full((16,), d, jnp.int32)
            av = plsc.load_gather(acc, [bidx, dvec])
            pos_p = carry[0] + av * plsc.load_gather(wb, [bidx, dvec])
            negs = tuple(
                carry[1 + n] + av * plsc.load_gather(nbufs[n], [bidx, dvec])
                for n in range(_NEG))
            return (pos_p,) + negs

        res = lax.fori_loop(0, _D, dot_step, (zeros,) * (1 + _NEG))
        linv = linv_v[pl.ds(g * 16, 16)]
        res_v[pl.ds(g * 16, 16)] = res[0] * linv
        for n in range(_NEG):
            m = mask_v[pl.ds(n * _BPW + g * 16, 16)]
            res_v[pl.ds((1 + n) * _BPW + g * 16, 16)] = res[1 + n] * linv * m

    pltpu.sync_copy(res_v.at[pl.ds(0, _BPW)], out_hbm.at[0, wid])
    for n in range(_NEG):
        pltpu.sync_copy(res_v.at[pl.ds((1 + n) * _BPW, _BPW)],
                        out_hbm.at[1 + n, wid])
        pltpu.sync_copy(mask_v.at[pl.ds(n * _BPW, _BPW)],
                        out_hbm.at[1 + _NEG + n, wid])


_sc_kernel = functools.partial(
    pl.kernel,
    out_type=jax.ShapeDtypeStruct((1 + 2 * _NEG, _NW, _BPW), jnp.float32),
    mesh=plsc.VectorSubcoreMesh(core_axis_name="c", subcore_axis_name="s"),
    compiler_params=pltpu.CompilerParams(
        needs_layout_passes=False, use_tc_tiling_on_sc=False),
    scratch_types=[
        pltpu.VMEM((_BPW, _NF), jnp.int32),
        pltpu.VMEM((2 * _W + 1 + _NEG, _BPW), jnp.int32),
        *[pltpu.VMEM((_BPW, _D), jnp.float32) for _ in range(11)],
        pltpu.VMEM((_BPW,), jnp.float32),
        pltpu.VMEM((_NEG * _BPW,), jnp.float32),
        pltpu.VMEM(((1 + _NEG) * _BPW,), jnp.float32),
        *[pltpu.SemaphoreType.DMA for _ in range(11)],
    ],
)(_sc_body)


def _loss_body(s_ref, o_ref):
    x = jnp.clip(s_ref[0], -10.0, 10.0)
    total = jnp.sum(jnp.log(1.0 + jnp.exp(-x)))
    for n in range(_NEG):
        z = jnp.clip(-s_ref[1 + n], -10.0, 10.0)
        total = total + jnp.sum(jnp.log(1.0 + jnp.exp(-z))
                                * s_ref[1 + _NEG + n])
    o_ref[...] = jnp.reshape(total, (1, 1))


def kernel(data, emb0, emb1):
    raw = _sc_kernel(data, emb0, emb1)
    loss = pl.pallas_call(
        _loss_body,
        out_shape=jax.ShapeDtypeStruct((1, 1), jnp.float32),
    )(raw)
    return loss[0, 0]


# speedup vs baseline: 3.2439x; 1.0285x over previous
import functools

import jax
import jax.numpy as jnp
from jax import lax
from jax.experimental import pallas as pl
from jax.experimental.pallas import tpu as pltpu
from jax.experimental.pallas import tpu_sc as plsc

_B = 4096
_V = 100000
_D = 64
_W = 5
_NEG = 5
_NW = 32
_BPW = _B // _NW
_NF = 2 * _W + 2 + 2 * _NEG
_NCH = _D // 16
_NG = _BPW // 16
_ECOLS = [2 * _W + 1] + list(range(2 * _W + 2, 2 * _W + 2 + _NEG))

_SC_PARAMS = pltpu.CompilerParams(
    needs_layout_passes=False, use_tc_tiling_on_sc=False)
_MESH = plsc.VectorSubcoreMesh(core_axis_name="c", subcore_axis_name="s")


def _worker_id():
    return lax.axis_index("s") * 2 + lax.axis_index("c")


def _stage_idx(d22, idxs, lane, cols):
    for f, col in enumerate(cols):
        cvec = jnp.full((16,), col, jnp.int32)
        for g in range(_NG):
            v = plsc.load_gather(d22, [lane + g * 16, cvec])
            idxs[f, pl.ds(g * 16, 16)] = v


def _ctx_body(data_hbm, emb0_hbm, acc_hbm,
              d22, idxs, acc, rb0, rb1, rb2, rb3,
              s_acc, s_r0, s_r1, s_r2, s_r3):
    wid = _worker_id()
    base = wid * _BPW
    lane = lax.iota(jnp.int32, 16)

    pltpu.sync_copy(data_hbm.at[pl.ds(base, _BPW)], d22)
    _stage_idx(d22, idxs, lane, list(range(2 * _W)))

    ring = [rb0, rb1, rb2, rb3]
    ring_sems = [s_r0, s_r1, s_r2, s_r3]
    cps = {0: pltpu.async_copy(emb0_hbm.at[idxs.at[0]], acc, s_acc)}
    for j in range(1, 5):
        cps[j] = pltpu.async_copy(
            emb0_hbm.at[idxs.at[j]], ring[j - 1], ring_sems[j - 1])

    cps[0].wait()
    for j in range(1, 2 * _W):
        slot = (j - 1) % 4
        buf = ring[slot]
        cps[j].wait()

        def add_body(i, _, buf=buf):
            for c in range(_NCH):
                sl = pl.ds(c * 16, 16)
                plsc.addupdate(acc.at[i, sl], buf[i, sl])
            return 0

        lax.fori_loop(0, _BPW, add_body, 0)
        nxt = j + 4
        if nxt < 2 * _W:
            cps[nxt] = pltpu.async_copy(
                emb0_hbm.at[idxs.at[nxt]], buf, ring_sems[slot])

    pltpu.sync_copy(acc, acc_hbm.at[wid])


_ctx_kernel = functools.partial(
    pl.kernel,
    out_type=jax.ShapeDtypeStruct((_NW, _BPW, _D), jnp.float32),
    mesh=_MESH,
    compiler_params=_SC_PARAMS,
    scratch_types=[
        pltpu.VMEM((_BPW, _NF), jnp.int32),
        pltpu.VMEM((2 * _W, _BPW), jnp.int32),
        *[pltpu.VMEM((_BPW, _D), jnp.float32) for _ in range(5)],
        *[pltpu.SemaphoreType.DMA for _ in range(5)],
    ],
)(_ctx_body)


def _dot_body(data_hbm, emb1_hbm, acc_hbm, out_hbm,
              d22, idxs, acc, wb, nb0, nb1, nb2, nb3, nb4,
              linv_v, mask_v, res_v,
              s_w, s_n0, s_n1, s_n2, s_n3, s_n4):
    wid = _worker_id()
    base = wid * _BPW
    lane = lax.iota(jnp.int32, 16)

    pltpu.sync_copy(data_hbm.at[pl.ds(base, _BPW)], d22)
    _stage_idx(d22, idxs, lane, _ECOLS)

    nbufs = [nb0, nb1, nb2, nb3, nb4]
    nsems = [s_n0, s_n1, s_n2, s_n3, s_n4]
    cw = pltpu.async_copy(emb1_hbm.at[idxs.at[0]], wb, s_w)
    cns = [
        pltpu.async_copy(emb1_hbm.at[idxs.at[1 + n]], nbufs[n], nsems[n])
        for n in range(_NEG)
    ]
    pltpu.sync_copy(acc_hbm.at[wid], acc)

    lcol = jnp.full((16,), 2 * _W, jnp.int32)
    one = jnp.full((16,), 1.0, jnp.float32)
    for g in range(_NG):
        lv = plsc.load_gather(d22, [lane + g * 16, lcol])
        linv_v[pl.ds(g * 16, 16)] = one / lv.astype(jnp.float32)
    for n in range(_NEG):
        mcol = jnp.full((16,), 2 * _W + 2 + _NEG + n, jnp.int32)
        for g in range(_NG):
            mv = plsc.load_gather(d22, [lane + g * 16, mcol])
            mask_v[pl.ds(n * _BPW + g * 16, 16)] = mv.astype(jnp.float32)

    cw.wait()
    for c in cns:
        c.wait()

    zeros = jnp.zeros((16,), jnp.float32)
    for g in range(_NG):
        bidx = lane + (g * 16)

        def dot_step(d, carry):
            dvec = jnp.full((16,), d, jnp.int32)
            av = plsc.load_gather(acc, [bidx, dvec])
            pos_p = carry[0] + av * plsc.load_gather(wb, [bidx, dvec])
            negs = tuple(
                carry[1 + n] + av * plsc.load_gather(nbufs[n], [bidx, dvec])
                for n in range(_NEG))
            return (pos_p,) + negs

        res = lax.fori_loop(0, _D, dot_step, (zeros,) * (1 + _NEG))
        linv = linv_v[pl.ds(g * 16, 16)]
        res_v[pl.ds(g * 16, 16)] = res[0] * linv
        for n in range(_NEG):
            m = mask_v[pl.ds(n * _BPW + g * 16, 16)]
            res_v[pl.ds((1 + n) * _BPW + g * 16, 16)] = res[1 + n] * linv * m

    pltpu.sync_copy(res_v.at[pl.ds(0, _BPW)], out_hbm.at[0, wid])
    for n in range(_NEG):
        pltpu.sync_copy(res_v.at[pl.ds((1 + n) * _BPW, _BPW)],
                        out_hbm.at[1 + n, wid])
        pltpu.sync_copy(mask_v.at[pl.ds(n * _BPW, _BPW)],
                        out_hbm.at[1 + _NEG + n, wid])


_dot_kernel = functools.partial(
    pl.kernel,
    out_type=jax.ShapeDtypeStruct((1 + 2 * _NEG, _NW, _BPW), jnp.float32),
    mesh=_MESH,
    compiler_params=_SC_PARAMS,
    scratch_types=[
        pltpu.VMEM((_BPW, _NF), jnp.int32),
        pltpu.VMEM((1 + _NEG, _BPW), jnp.int32),
        *[pltpu.VMEM((_BPW, _D), jnp.float32) for _ in range(7)],
        pltpu.VMEM((_BPW,), jnp.float32),
        pltpu.VMEM((_NEG * _BPW,), jnp.float32),
        pltpu.VMEM(((1 + _NEG) * _BPW,), jnp.float32),
        *[pltpu.SemaphoreType.DMA for _ in range(6)],
    ],
)(_dot_body)


def _loss_body(s_ref, o_ref):
    x = jnp.clip(s_ref[0], -10.0, 10.0)
    total = jnp.sum(jnp.log(1.0 + jnp.exp(-x)))
    for n in range(_NEG):
        z = jnp.clip(-s_ref[1 + n], -10.0, 10.0)
        total = total + jnp.sum(jnp.log(1.0 + jnp.exp(-z))
                                * s_ref[1 + _NEG + n])
    o_ref[...] = jnp.reshape(total, (1, 1))


def kernel(data, emb0, emb1):
    ctx_sums = _ctx_kernel(data, emb0)
    raw = _dot_kernel(data, emb1, ctx_sums)
    loss = pl.pallas_call(
        _loss_body,
        out_shape=jax.ShapeDtypeStruct((1, 1), jnp.float32),
    )(raw)
    return loss[0, 0]
